# baseline (speedup 1.0000x reference)
import jax
import jax.numpy as jnp
from jax.experimental import pallas as pl
from jax.experimental.pallas import tpu as pltpu

B = 2
N = 4096
D = 256
BI = 512
RB = 512


def _top2max_tournament(v_ref):
    r = N // 2
    t1 = jnp.maximum(v_ref[:r, :], v_ref[r:, :])
    t2 = jnp.minimum(v_ref[:r, :], v_ref[r:, :])
    while r > 1:
        h = r // 2
        a1, b1 = t1[:h], t1[h:]
        a2, b2 = t2[:h], t2[h:]
        t1 = jnp.maximum(a1, b1)
        t2 = jnp.maximum(jnp.minimum(a1, b1), jnp.maximum(a2, b2))
        r = h
    return t1, t2


def _knn2_kernel(xi_ref, xj_ref, out_ref, kaug_ref, v_ref):
    i = pl.program_id(1)

    @pl.when(i == 0)
    def _build_keys():
        xj = xj_ref[0]
        kaug_ref[:, 0:D] = xj
        kaug_ref[:, D : D + 1] = 0.5 * jnp.sum(xj * xj, axis=1, keepdims=True)

    xi = xi_ref[0]
    qaug = jnp.concatenate(
        [xi, jnp.full((BI, 1), -1.0, jnp.float32)], axis=1
    )
    v_ref[...] = jax.lax.dot_general(
        kaug_ref[...], qaug, (((1,), (1,)), ((), ())),
        preferred_element_type=jnp.float32,
    )
    sub = v_ref[pl.ds(i * BI, BI), :]
    diag = (
        jax.lax.broadcasted_iota(jnp.int32, (BI, BI), 0)
        == jax.lax.broadcasted_iota(jnp.int32, (BI, BI), 1)
    )
    v_ref[pl.ds(i * BI, BI), :] = jnp.where(diag, -jnp.inf, sub)
    m1, m2 = _top2max_tournament(v_ref)
    sqi = 2.0 * jnp.transpose(kaug_ref[pl.ds(i * BI, BI), D : D + 1])
    out_ref[0, 0:1, :] = jnp.maximum(sqi - 2.0 * m1, 0.0)
    out_ref[0, 1:2, :] = jnp.maximum(sqi - 2.0 * m2, 0.0)


def _twonn_kernel(dfull_ref, o1_ref, o2_ref):
    i = pl.program_id(1)
    d1f = dfull_ref[0, 0:1, :]
    d2f = dfull_ref[0, 1:2, :]
    tf = 0.5 * (jnp.log(d2f) - jnp.log(d1f))
    d1s = dfull_ref[0, 0:1, pl.ds(i * RB, RB)]
    d2s = dfull_ref[0, 1:2, pl.ds(i * RB, RB)]
    tb = jnp.transpose(
        0.5 * (jnp.log(d2s) - jnp.log(d1s))
    )
    less = (tf < tb).astype(jnp.float32)
    rank = jnp.sum(less, axis=1, keepdims=True)
    y = jnp.log(jnp.float32(N)) - jnp.log(jnp.float32(N) - rank)
    sxy = jnp.sum(tb * y)
    sxx = jnp.sum(tb * tb)
    first = i == 0
    o1_ref[...] = jnp.where(first, 0.0, o1_ref[...]) + sxy
    o2_ref[...] = jnp.where(first, 0.0, o2_ref[...]) + sxx


def kernel(X):
    d12 = pl.pallas_call(
        _knn2_kernel,
        grid=(B, N // BI),
        in_specs=[
            pl.BlockSpec((1, BI, D), lambda b, i: (b, i, 0)),
            pl.BlockSpec((1, N, D), lambda b, i: (b, 0, 0)),
        ],
        out_specs=pl.BlockSpec((1, 2, BI), lambda b, i: (b, 0, i)),
        out_shape=jax.ShapeDtypeStruct((B, 2, N), jnp.float32),
        scratch_shapes=[
            pltpu.VMEM((N, D + 1), jnp.float32),
            pltpu.VMEM((N, BI), jnp.float32),
        ],
        compiler_params=pltpu.CompilerParams(
            dimension_semantics=("parallel", "arbitrary"),
        ),
    )(X, X)
    o1, o2 = pl.pallas_call(
        _twonn_kernel,
        grid=(B, N // RB),
        in_specs=[
            pl.BlockSpec((1, 2, N), lambda b, i: (b, 0, 0)),
        ],
        out_specs=[
            pl.BlockSpec((1, 8, 128), lambda b, i: (b, 0, 0)),
            pl.BlockSpec((1, 8, 128), lambda b, i: (b, 0, 0)),
        ],
        out_shape=[
            jax.ShapeDtypeStruct((B, 8, 128), jnp.float32),
            jax.ShapeDtypeStruct((B, 8, 128), jnp.float32),
        ],
        compiler_params=pltpu.CompilerParams(
            dimension_semantics=("parallel", "arbitrary"),
        ),
    )(d12)
    return o1[:, 0, 0] / o2[:, 0, 0]


# speedup vs baseline: 1.6223x; 1.6223x over previous
import jax
import jax.numpy as jnp
from jax.experimental import pallas as pl
from jax.experimental.pallas import tpu as pltpu

B = 2
N = 4096
D = 256
BI = 512
RB = 512


def _top3_tournament(s):
    r = s.shape[0] // 2
    t1 = jnp.minimum(s[:r], s[r:])
    t2 = jnp.maximum(s[:r], s[r:])
    r //= 2
    a1, b1 = t1[:r], t1[r:]
    a2, b2 = t2[:r], t2[r:]
    mx1 = jnp.maximum(a1, b1)
    mn2 = jnp.minimum(a2, b2)
    t1 = jnp.minimum(a1, b1)
    t3 = jnp.maximum(mx1, mn2)
    t2 = jnp.minimum(mx1, mn2)
    while r > 1:
        r //= 2
        a1, b1 = t1[:r], t1[r:]
        a2, b2 = t2[:r], t2[r:]
        a3, b3 = t3[:r], t3[r:]
        mx1 = jnp.maximum(a1, b1)
        mn2 = jnp.minimum(a2, b2)
        mx2 = jnp.maximum(a2, b2)
        mn3 = jnp.minimum(a3, b3)
        t1 = jnp.minimum(a1, b1)
        t2 = jnp.minimum(mx1, mn2)
        t3 = jnp.minimum(jnp.maximum(mx1, mn2), jnp.minimum(mx2, mn3))
    return t1, t2, t3


def _knn2_kernel(xi_ref, xj_ref, out_ref, sqjh_ref):
    @pl.when(pl.program_id(1) == 0)
    def _norms():
        xj = xj_ref[0]
        sqjh_ref[:, 0] = 0.5 * jnp.sum(xj * xj, axis=1)

    xi = xi_ref[0]
    xj = xj_ref[0]
    dot = jax.lax.dot_general(
        xj, xi, (((1,), (1,)), ((), ())), preferred_element_type=jnp.float32
    )
    s = sqjh_ref[...] - dot
    t1, m2, m3 = _top3_tournament(s)
    out_ref[0, 0:1, :] = 2.0 * (m2 - t1)
    out_ref[0, 1:2, :] = 2.0 * (m3 - t1)


def _twonn_kernel(dfull_ref, o1_ref, o2_ref):
    i = pl.program_id(1)
    d1f = dfull_ref[0, 0:1, :]
    d2f = dfull_ref[0, 1:2, :]
    tf = 0.5 * (jnp.log(d2f) - jnp.log(d1f))
    d1s = dfull_ref[0, 0:1, pl.ds(i * RB, RB)]
    d2s = dfull_ref[0, 1:2, pl.ds(i * RB, RB)]
    tb = jnp.transpose(
        0.5 * (jnp.log(d2s) - jnp.log(d1s))
    )
    rank = jnp.count_nonzero(tf < tb, axis=1, keepdims=True).astype(
        jnp.float32
    )
    y = jnp.log(jnp.float32(N)) - jnp.log(jnp.float32(N) - rank)
    sxy = jnp.sum(tb * y)
    sxx = jnp.sum(tb * tb)
    first = i == 0
    o1_ref[...] = jnp.where(first, 0.0, o1_ref[...]) + sxy
    o2_ref[...] = jnp.where(first, 0.0, o2_ref[...]) + sxx


def kernel(X):
    d12 = pl.pallas_call(
        _knn2_kernel,
        grid=(B, N // BI),
        in_specs=[
            pl.BlockSpec((1, BI, D), lambda b, i: (b, i, 0)),
            pl.BlockSpec((1, N, D), lambda b, i: (b, 0, 0)),
        ],
        out_specs=pl.BlockSpec((1, 2, BI), lambda b, i: (b, 0, i)),
        out_shape=jax.ShapeDtypeStruct((B, 2, N), jnp.float32),
        scratch_shapes=[pltpu.VMEM((N, 1), jnp.float32)],
        compiler_params=pltpu.CompilerParams(
            dimension_semantics=("parallel", "arbitrary"),
        ),
    )(X, X)
    o1, o2 = pl.pallas_call(
        _twonn_kernel,
        grid=(B, N // RB),
        in_specs=[
            pl.BlockSpec((1, 2, N), lambda b, i: (b, 0, 0)),
        ],
        out_specs=[
            pl.BlockSpec((1, 8, 128), lambda b, i: (b, 0, 0)),
            pl.BlockSpec((1, 8, 128), lambda b, i: (b, 0, 0)),
        ],
        out_shape=[
            jax.ShapeDtypeStruct((B, 8, 128), jnp.float32),
            jax.ShapeDtypeStruct((B, 8, 128), jnp.float32),
        ],
        compiler_params=pltpu.CompilerParams(
            dimension_semantics=("parallel", "arbitrary"),
        ),
    )(d12)
    return o1[:, 0, 0] / o2[:, 0, 0]


# speedup vs baseline: 1.7433x; 1.0746x over previous
import jax
import jax.numpy as jnp
from jax.experimental import pallas as pl
from jax.experimental.pallas import tpu as pltpu

B = 2
N = 4096
D = 256
BI = 512
RB = 512
NI = N // BI


def _top3_tournament(s):
    r = s.shape[0] // 2
    t1 = jnp.minimum(s[:r], s[r:])
    t2 = jnp.maximum(s[:r], s[r:])
    r //= 2
    a1, b1 = t1[:r], t1[r:]
    a2, b2 = t2[:r], t2[r:]
    mx1 = jnp.maximum(a1, b1)
    mn2 = jnp.minimum(a2, b2)
    t1 = jnp.minimum(a1, b1)
    t3 = jnp.maximum(mx1, mn2)
    t2 = jnp.minimum(mx1, mn2)
    while r > 1:
        r //= 2
        a1, b1 = t1[:r], t1[r:]
        a2, b2 = t2[:r], t2[r:]
        a3, b3 = t3[:r], t3[r:]
        mx1 = jnp.maximum(a1, b1)
        mn2 = jnp.minimum(a2, b2)
        mx2 = jnp.maximum(a2, b2)
        mn3 = jnp.minimum(a3, b3)
        t1 = jnp.minimum(a1, b1)
        t2 = jnp.minimum(mx1, mn2)
        t3 = jnp.minimum(jnp.maximum(mx1, mn2), jnp.minimum(mx2, mn3))
    return t1, t2, t3


def _twonn_fused_kernel(xi_ref, xj_ref, o1_ref, o2_ref, sqjh_ref, d_ref):
    i = pl.program_id(1)

    @pl.when(i == 0)
    def _norms():
        xj = xj_ref[0]
        sqjh_ref[:, 0] = 0.5 * jnp.sum(xj * xj, axis=1)

    xi = xi_ref[0]
    xj = xj_ref[0]
    dot = jax.lax.dot_general(
        xj, xi, (((1,), (1,)), ((), ())), preferred_element_type=jnp.float32
    )
    s = sqjh_ref[...] - dot
    t1, m2, m3 = _top3_tournament(s)
    d_ref[0:1, pl.ds(i * BI, BI)] = 2.0 * (m2 - t1)
    d_ref[1:2, pl.ds(i * BI, BI)] = 2.0 * (m3 - t1)

    @pl.when(i == NI - 1)
    def _twonn():
        d1f = d_ref[0:1, :]
        d2f = d_ref[1:2, :]
        tf = 0.5 * (jnp.log(d2f) - jnp.log(d1f))
        sxy = jnp.float32(0.0)
        sxx = jnp.sum(tf * tf)
        for r in range(N // RB):
            tb = jnp.transpose(tf[:, r * RB : (r + 1) * RB])
            rank = jnp.count_nonzero(tf < tb, axis=1, keepdims=True).astype(
                jnp.float32
            )
            y = jnp.log(jnp.float32(N)) - jnp.log(jnp.float32(N) - rank)
            sxy = sxy + jnp.sum(tb * y)
        o1_ref[...] = jnp.full((1, 8, 128), 1.0, jnp.float32) * sxy
        o2_ref[...] = jnp.full((1, 8, 128), 1.0, jnp.float32) * sxx


def kernel(X):
    o1, o2 = pl.pallas_call(
        _twonn_fused_kernel,
        grid=(B, NI),
        in_specs=[
            pl.BlockSpec((1, BI, D), lambda b, i: (b, i, 0)),
            pl.BlockSpec((1, N, D), lambda b, i: (b, 0, 0)),
        ],
        out_specs=[
            pl.BlockSpec((1, 8, 128), lambda b, i: (b, 0, 0)),
            pl.BlockSpec((1, 8, 128), lambda b, i: (b, 0, 0)),
        ],
        out_shape=[
            jax.ShapeDtypeStruct((B, 8, 128), jnp.float32),
            jax.ShapeDtypeStruct((B, 8, 128), jnp.float32),
        ],
        scratch_shapes=[
            pltpu.VMEM((N, 1), jnp.float32),
            pltpu.VMEM((2, N), jnp.float32),
        ],
        compiler_params=pltpu.CompilerParams(
            dimension_semantics=("parallel", "arbitrary"),
        ),
    )(X, X)
    return o1[:, 0, 0] / o2[:, 0, 0]
